# Initial kernel scaffold; baseline (speedup 1.0000x reference)
import jax
import jax.numpy as jnp
from jax.experimental import pallas as pl


def kernel(x, edge_index, W1, b1, W2, b2):
    raise NotImplementedError("write your pallas kernel here")


# speedup vs baseline: 3.8885x; 3.8885x over previous
import functools

import jax
import jax.numpy as jnp
from jax import lax
from jax.experimental import pallas as pl
from jax.experimental.pallas import tpu as pltpu
from jax.experimental.pallas import tpu_sc as plsc

N = 10000
E = 320000
D = 128

NC = 2
NS = 16
NW = NC * NS

EDGES_PER_W = E // NW
CHUNK = 80
NCHUNKS = EDGES_PER_W // CHUNK
SELF_PER_W = 320


def _mlp_body(x_ref, w1_ref, b1_ref, w2_ref, b2_ref, o_ref):
    w1c = w1_ref[:, :D] + w1_ref[:, D:]
    h = lax.dot_general(x_ref[...], w1c, (((1,), (1,)), ((), ())),
                        preferred_element_type=jnp.float32)
    h = jnp.maximum(h + b1_ref[...], 0.0)
    o = lax.dot_general(h, w2_ref[...], (((1,), (1,)), ((), ())),
                        preferred_element_type=jnp.float32)
    o_ref[...] = jnp.maximum(o + b2_ref[...], 0.0)


def _node_mlp(x, w1, b1, w2, b2):
    blk = 2000
    grid = N // blk
    return pl.pallas_call(
        _mlp_body,
        grid=(grid,),
        in_specs=[
            pl.BlockSpec((blk, D), lambda i: (i, 0)),
            pl.BlockSpec((D, 2 * D), lambda i: (0, 0)),
            pl.BlockSpec((1, D), lambda i: (0, 0)),
            pl.BlockSpec((D, D), lambda i: (0, 0)),
            pl.BlockSpec((1, D), lambda i: (0, 0)),
        ],
        out_specs=pl.BlockSpec((blk, D), lambda i: (i, 0)),
        out_shape=jax.ShapeDtypeStruct((N, D), jnp.float32),
    )(x, w1, b1.reshape(1, D), w2, b2.reshape(1, D))


def _sc_gather(y, idx3):
    mesh = plsc.VectorSubcoreMesh(core_axis_name="c", subcore_axis_name="s")

    @functools.partial(
        pl.kernel,
        mesh=mesh,
        out_type=jax.ShapeDtypeStruct((N + E, D), jnp.float32),
        scratch_types=[
            pltpu.VMEM((NCHUNKS, CHUNK), jnp.int32),
            pltpu.VMEM((CHUNK, D), jnp.float32),
            pltpu.VMEM((SELF_PER_W, D), jnp.float32),
            pltpu.SemaphoreType.DMA,
        ],
    )
    def gather_kernel(y_hbm, idx_hbm, out_hbm, idx_v, rows_v, self_v, gsem):
        wid = lax.axis_index("s") * NC + lax.axis_index("c")

        base = jnp.minimum(wid * SELF_PER_W, N - SELF_PER_W)
        pltpu.sync_copy(y_hbm.at[pl.ds(base, SELF_PER_W)], self_v)
        pltpu.sync_copy(self_v, out_hbm.at[pl.ds(base, SELF_PER_W)])

        pltpu.sync_copy(idx_hbm.at[wid], idx_v)

        dst0 = N + wid * EDGES_PER_W

        def step(j, carry):
            pltpu.async_copy(y_hbm.at[idx_v.at[j]], rows_v, gsem).wait()
            pltpu.sync_copy(rows_v, out_hbm.at[pl.ds(dst0 + j * CHUNK, CHUNK)])
            return carry

        lax.fori_loop(0, NCHUNKS, step, 0)

    return gather_kernel(y, idx3)


def kernel(x, edge_index, W1, b1, W2, b2):
    y = _node_mlp(x, W1, b1, W2, b2)
    idx3 = edge_index[0].reshape(NW, NCHUNKS, CHUNK)
    return _sc_gather(y, idx3)


# speedup vs baseline: 5.6284x; 1.4474x over previous
import functools

import jax
import jax.numpy as jnp
from jax import lax
from jax.experimental import pallas as pl
from jax.experimental.pallas import tpu as pltpu
from jax.experimental.pallas import tpu_sc as plsc

N = 10000
E = 320000
D = 128

NC = 2
NS = 16
NW = NC * NS

EDGES_PER_W = E // NW
CHUNK = 80
NCHUNKS = EDGES_PER_W // CHUNK
SELF_PER_W = 320


def _mlp_body(x_ref, w1_ref, b1_ref, w2_ref, b2_ref, o_ref):
    w1c = w1_ref[:, :D] + w1_ref[:, D:]
    h = lax.dot_general(x_ref[...], w1c, (((1,), (1,)), ((), ())),
                        preferred_element_type=jnp.float32)
    h = jnp.maximum(h + b1_ref[...], 0.0)
    o = lax.dot_general(h, w2_ref[...], (((1,), (1,)), ((), ())),
                        preferred_element_type=jnp.float32)
    o_ref[...] = jnp.maximum(o + b2_ref[...], 0.0)


def _node_mlp(x, w1, b1, w2, b2):
    blk = 2000
    grid = N // blk
    return pl.pallas_call(
        _mlp_body,
        grid=(grid,),
        in_specs=[
            pl.BlockSpec((blk, D), lambda i: (i, 0)),
            pl.BlockSpec((D, 2 * D), lambda i: (0, 0)),
            pl.BlockSpec((1, D), lambda i: (0, 0)),
            pl.BlockSpec((D, D), lambda i: (0, 0)),
            pl.BlockSpec((1, D), lambda i: (0, 0)),
        ],
        out_specs=pl.BlockSpec((blk, D), lambda i: (i, 0)),
        out_shape=jax.ShapeDtypeStruct((N, D), jnp.float32),
    )(x, w1, b1.reshape(1, D), w2, b2.reshape(1, D))


def _sc_gather(y, idx3):
    mesh = plsc.VectorSubcoreMesh(core_axis_name="c", subcore_axis_name="s")

    @functools.partial(
        pl.kernel,
        mesh=mesh,
        out_type=jax.ShapeDtypeStruct((N + E, D), jnp.float32),
        scratch_types=[
            pltpu.VMEM((NCHUNKS, CHUNK), jnp.int32),
            pltpu.VMEM((CHUNK, D), jnp.float32),
            pltpu.VMEM((CHUNK, D), jnp.float32),
            pltpu.VMEM((SELF_PER_W, D), jnp.float32),
            pltpu.SemaphoreType.DMA,
            pltpu.SemaphoreType.DMA,
            pltpu.SemaphoreType.DMA,
            pltpu.SemaphoreType.DMA,
        ],
    )
    def gather_kernel(y_hbm, idx_hbm, out_hbm, idx_v, rows0, rows1,
                      self_v, gsem0, gsem1, wsem0, wsem1):
        wid = lax.axis_index("s") * NC + lax.axis_index("c")

        base = jnp.minimum(wid * SELF_PER_W, N - SELF_PER_W)
        pltpu.sync_copy(y_hbm.at[pl.ds(base, SELF_PER_W)], self_v)
        pltpu.sync_copy(self_v, out_hbm.at[pl.ds(base, SELF_PER_W)])

        pltpu.sync_copy(idx_hbm.at[wid], idx_v)

        dst0 = N + wid * EDGES_PER_W
        rows = (rows0, rows1)
        gsem = (gsem0, gsem1)
        wsem = (wsem0, wsem1)

        def out_at(j):
            return out_hbm.at[pl.ds(dst0 + j * CHUNK, CHUNK)]

        pltpu.async_copy(y_hbm.at[idx_v.at[0]], rows0, gsem0)

        def step(j, carry):
            for b in range(2):
                @pl.when(lax.rem(j, 2) == b)
                def _():
                    o = 1 - b
                    @pl.when(j >= 1)
                    def _():
                        pltpu.make_async_copy(
                            rows[o], out_at(j - 1), wsem[o]).wait()
                    @pl.when(j + 1 < NCHUNKS)
                    def _():
                        pltpu.async_copy(
                            y_hbm.at[idx_v.at[j + 1]], rows[o], gsem[o])
                    pltpu.make_async_copy(
                        y_hbm.at[idx_v.at[j]], rows[b], gsem[b]).wait()
                    pltpu.async_copy(rows[b], out_at(j), wsem[b])
            return carry

        lax.fori_loop(0, NCHUNKS, step, 0)
        last = NCHUNKS - 1
        pltpu.make_async_copy(rows[last % 2], out_at(last),
                              wsem[last % 2]).wait()

    return gather_kernel(y, idx3)


def kernel(x, edge_index, W1, b1, W2, b2):
    y = _node_mlp(x, W1, b1, W2, b2)
    idx3 = edge_index[0].reshape(NW, NCHUNKS, CHUNK)
    return _sc_gather(y, idx3)


# speedup vs baseline: 6.0566x; 1.0761x over previous
import functools

import jax
import jax.numpy as jnp
from jax import lax
from jax.experimental import pallas as pl
from jax.experimental.pallas import tpu as pltpu
from jax.experimental.pallas import tpu_sc as plsc

N = 10000
E = 320000
D = 128

NC = 2
NS = 16
NW = NC * NS

EDGES_PER_W = E // NW
CHUNK = 128
NCHUNKS = -(-EDGES_PER_W // CHUNK)
SELF_PER_W = 320


def _mlp_body(x_ref, w1_ref, b1_ref, w2_ref, b2_ref, o_ref):
    w1c = w1_ref[:, :D] + w1_ref[:, D:]
    h = lax.dot_general(x_ref[...], w1c, (((1,), (1,)), ((), ())),
                        preferred_element_type=jnp.float32)
    h = jnp.maximum(h + b1_ref[...], 0.0)
    o = lax.dot_general(h, w2_ref[...], (((1,), (1,)), ((), ())),
                        preferred_element_type=jnp.float32)
    o_ref[...] = jnp.maximum(o + b2_ref[...], 0.0)


def _node_mlp(x, w1, b1, w2, b2):
    blk = 2000
    grid = N // blk
    return pl.pallas_call(
        _mlp_body,
        grid=(grid,),
        in_specs=[
            pl.BlockSpec((blk, D), lambda i: (i, 0)),
            pl.BlockSpec((D, 2 * D), lambda i: (0, 0)),
            pl.BlockSpec((1, D), lambda i: (0, 0)),
            pl.BlockSpec((D, D), lambda i: (0, 0)),
            pl.BlockSpec((1, D), lambda i: (0, 0)),
        ],
        out_specs=pl.BlockSpec((blk, D), lambda i: (i, 0)),
        out_shape=jax.ShapeDtypeStruct((N, D), jnp.float32),
    )(x, w1, b1.reshape(1, D), w2, b2.reshape(1, D))


def _sc_gather(y, idx3):
    mesh = plsc.VectorSubcoreMesh(core_axis_name="c", subcore_axis_name="s")

    @functools.partial(
        pl.kernel,
        mesh=mesh,
        out_type=jax.ShapeDtypeStruct((N + E, D), jnp.float32),
        scratch_types=[
            pltpu.VMEM((NCHUNKS, CHUNK), jnp.int32),
            pltpu.VMEM((CHUNK, D), jnp.float32),
            pltpu.VMEM((CHUNK, D), jnp.float32),
            pltpu.VMEM((SELF_PER_W, D), jnp.float32),
            pltpu.SemaphoreType.DMA,
            pltpu.SemaphoreType.DMA,
            pltpu.SemaphoreType.DMA,
            pltpu.SemaphoreType.DMA,
            pltpu.SemaphoreType.DMA,
        ],
    )
    def gather_kernel(y_hbm, idx_hbm, out_hbm, idx_v, rows0, rows1,
                      self_v, gsem0, gsem1, wsem0, wsem1, ssem):
        wid = lax.axis_index("s") * NC + lax.axis_index("c")

        base = jnp.minimum(wid * SELF_PER_W, N - SELF_PER_W)
        pltpu.async_copy(y_hbm.at[pl.ds(base, SELF_PER_W)], self_v, ssem)

        pltpu.sync_copy(idx_hbm.at[wid], idx_v)

        dst0 = N + wid * EDGES_PER_W
        rows = (rows0, rows1)
        gsem = (gsem0, gsem1)
        wsem = (wsem0, wsem1)

        def out_at(j):
            off = jnp.minimum(j * CHUNK, EDGES_PER_W - CHUNK)
            return out_hbm.at[pl.ds(dst0 + off, CHUNK)]

        pltpu.async_copy(y_hbm.at[idx_v.at[0]], rows0, gsem0)

        pltpu.make_async_copy(
            y_hbm.at[pl.ds(base, SELF_PER_W)], self_v, ssem).wait()
        pltpu.async_copy(self_v, out_hbm.at[pl.ds(base, SELF_PER_W)], ssem)

        def step(j, carry):
            for b in range(2):
                @pl.when(lax.rem(j, 2) == b)
                def _():
                    o = 1 - b
                    @pl.when(j >= 1)
                    def _():
                        pltpu.make_async_copy(
                            rows[o], out_at(j - 1), wsem[o]).wait()
                    @pl.when(j + 1 < NCHUNKS)
                    def _():
                        pltpu.async_copy(
                            y_hbm.at[idx_v.at[j + 1]], rows[o], gsem[o])
                    pltpu.make_async_copy(
                        y_hbm.at[idx_v.at[j]], rows[b], gsem[b]).wait()
                    pltpu.async_copy(rows[b], out_at(j), wsem[b])
            return carry

        lax.fori_loop(0, NCHUNKS, step, 0)
        last = NCHUNKS - 1
        pltpu.make_async_copy(rows[last % 2], out_at(last),
                              wsem[last % 2]).wait()
        pltpu.make_async_copy(
            self_v, out_hbm.at[pl.ds(base, SELF_PER_W)], ssem).wait()

    return gather_kernel(y, idx3)


def kernel(x, edge_index, W1, b1, W2, b2):
    y = _node_mlp(x, W1, b1, W2, b2)
    src = edge_index[0].reshape(NW, EDGES_PER_W)
    main = src[:, :(NCHUNKS - 1) * CHUNK].reshape(NW, NCHUNKS - 1, CHUNK)
    tail = src[:, EDGES_PER_W - CHUNK:].reshape(NW, 1, CHUNK)
    idx3 = jnp.concatenate([main, tail], axis=1)
    return _sc_gather(y, idx3)


# speedup vs baseline: 6.1575x; 1.0167x over previous
import functools

import jax
import jax.numpy as jnp
from jax import lax
from jax.experimental import pallas as pl
from jax.experimental.pallas import tpu as pltpu
from jax.experimental.pallas import tpu_sc as plsc

N = 10000
E = 320000
D = 128

NC = 2
NS = 16
NW = NC * NS

EDGES_PER_W = E // NW
CHUNK = 128
NCHUNKS = -(-EDGES_PER_W // CHUNK)
SELF_PER_W = 320
NBUF = 4
PREF = 2


def _mlp_body(x_ref, w1_ref, b1_ref, w2_ref, b2_ref, o_ref):
    w1c = w1_ref[:, :D] + w1_ref[:, D:]
    h = lax.dot_general(x_ref[...], w1c, (((1,), (1,)), ((), ())),
                        preferred_element_type=jnp.float32)
    h = jnp.maximum(h + b1_ref[...], 0.0)
    o = lax.dot_general(h, w2_ref[...], (((1,), (1,)), ((), ())),
                        preferred_element_type=jnp.float32)
    o_ref[...] = jnp.maximum(o + b2_ref[...], 0.0)


def _node_mlp(x, w1, b1, w2, b2):
    blk = 2000
    grid = N // blk
    return pl.pallas_call(
        _mlp_body,
        grid=(grid,),
        in_specs=[
            pl.BlockSpec((blk, D), lambda i: (i, 0)),
            pl.BlockSpec((D, 2 * D), lambda i: (0, 0)),
            pl.BlockSpec((1, D), lambda i: (0, 0)),
            pl.BlockSpec((D, D), lambda i: (0, 0)),
            pl.BlockSpec((1, D), lambda i: (0, 0)),
        ],
        out_specs=pl.BlockSpec((blk, D), lambda i: (i, 0)),
        out_shape=jax.ShapeDtypeStruct((N, D), jnp.float32),
    )(x, w1, b1.reshape(1, D), w2, b2.reshape(1, D))


def _sc_gather(y, idx3):
    mesh = plsc.VectorSubcoreMesh(core_axis_name="c", subcore_axis_name="s")

    @functools.partial(
        pl.kernel,
        mesh=mesh,
        out_type=jax.ShapeDtypeStruct((N + E, D), jnp.float32),
        scratch_types=(
            [pltpu.VMEM((NCHUNKS, CHUNK), jnp.int32)]
            + [pltpu.VMEM((CHUNK, D), jnp.float32) for _ in range(NBUF)]
            + [pltpu.VMEM((SELF_PER_W, D), jnp.float32)]
            + [pltpu.SemaphoreType.DMA for _ in range(2 * NBUF + 1)]
        ),
    )
    def gather_kernel(y_hbm, idx_hbm, out_hbm, idx_v, *scratch):
        rows = scratch[:NBUF]
        self_v = scratch[NBUF]
        gsem = scratch[NBUF + 1:2 * NBUF + 1]
        wsem = scratch[2 * NBUF + 1:3 * NBUF + 1]
        ssem = scratch[3 * NBUF + 1]
        wid = lax.axis_index("s") * NC + lax.axis_index("c")

        base = jnp.minimum(wid * SELF_PER_W, N - SELF_PER_W)
        pltpu.async_copy(y_hbm.at[pl.ds(base, SELF_PER_W)], self_v, ssem)

        pltpu.sync_copy(idx_hbm.at[wid], idx_v)

        dst0 = N + wid * EDGES_PER_W

        def out_at(j):
            off = jnp.minimum(j * CHUNK, EDGES_PER_W - CHUNK)
            return out_hbm.at[pl.ds(dst0 + off, CHUNK)]

        for jj in range(PREF):
            pltpu.async_copy(y_hbm.at[idx_v.at[jj]], rows[jj], gsem[jj])

        pltpu.make_async_copy(
            y_hbm.at[pl.ds(base, SELF_PER_W)], self_v, ssem).wait()
        pltpu.async_copy(self_v, out_hbm.at[pl.ds(base, SELF_PER_W)], ssem)

        def step(j, carry):
            for b in range(NBUF):
                @pl.when(lax.rem(j, NBUF) == b)
                def _():
                    nb = (b + PREF) % NBUF
                    @pl.when(j + PREF >= NBUF)
                    def _():
                        pltpu.make_async_copy(
                            rows[nb], out_at(j + PREF - NBUF),
                            wsem[nb]).wait()
                    @pl.when(j + PREF < NCHUNKS)
                    def _():
                        pltpu.async_copy(
                            y_hbm.at[idx_v.at[j + PREF]], rows[nb],
                            gsem[nb])
                    pltpu.make_async_copy(
                        y_hbm.at[idx_v.at[j]], rows[b], gsem[b]).wait()
                    pltpu.async_copy(rows[b], out_at(j), wsem[b])
            return carry

        lax.fori_loop(0, NCHUNKS, step, 0)
        for j in range(NCHUNKS - (NBUF - PREF), NCHUNKS):
            pltpu.make_async_copy(rows[j % NBUF], out_at(j),
                                  wsem[j % NBUF]).wait()
        pltpu.make_async_copy(
            self_v, out_hbm.at[pl.ds(base, SELF_PER_W)], ssem).wait()

    return gather_kernel(y, idx3)


def kernel(x, edge_index, W1, b1, W2, b2):
    y = _node_mlp(x, W1, b1, W2, b2)
    src = edge_index[0].reshape(NW, EDGES_PER_W)
    main = src[:, :(NCHUNKS - 1) * CHUNK].reshape(NW, NCHUNKS - 1, CHUNK)
    tail = src[:, EDGES_PER_W - CHUNK:].reshape(NW, 1, CHUNK)
    idx3 = jnp.concatenate([main, tail], axis=1)
    return _sc_gather(y, idx3)
